# Initial kernel scaffold; baseline (speedup 1.0000x reference)
import jax
import jax.numpy as jnp
from jax.experimental import pallas as pl


def kernel(x, table):
    raise NotImplementedError("write your pallas kernel here")


# speedup vs baseline: 2.3328x; 2.3328x over previous
import functools

import jax
import jax.numpy as jnp
from jax import lax
from jax.experimental import pallas as pl
from jax.experimental.pallas import tpu as pltpu
from jax.experimental.pallas import tpu_sc as plsc

NC = 2
NS = 16
NW = NC * NS
L = 16

B = 16384
HL = 50
D = 32
BPW = B // NW
NCH = D // L


def _accum(acc, buf):
    for l in range(HL):
        for c in range(NCH):
            sl = pl.ds(c * L, L)
            acc[l, sl] = acc[l, sl] + buf[l, sl]


def _sc_body(x_hbm, table_hbm, out_hbm, idx_v, buf0, buf1, acc, sidx, sem0, sem1,
             shared):
    cid = lax.axis_index("c")
    sid = lax.axis_index("s")
    wid = sid * NC + cid
    base = wid * BPW

    pltpu.sync_copy(x_hbm.at[pl.ds(base, BPW)], idx_v)

    zero = jnp.zeros((L,), jnp.float32)
    for l in range(64):
        for c in range(NCH):
            acc[l, pl.ds(c * L, L)] = zero

    iota = lax.iota(jnp.int32, L)
    for c in range(4):
        v = iota + (c * L)
        sidx[pl.ds(c * L, L)] = jnp.where(v < HL, v, 0)

    @pl.when(sid == 0)
    def _():
        pltpu.sync_copy(acc.at[pl.ds(0, HL + 1)], shared)

    plsc.subcore_barrier()

    pltpu.async_copy(table_hbm.at[idx_v.at[0]], buf0, sem0)
    pltpu.async_copy(table_hbm.at[idx_v.at[1]], buf1, sem1)

    def step(i, _):
        r = 2 * i
        pltpu.make_async_copy(table_hbm.at[idx_v.at[0]], buf0, sem0).wait()
        _accum(acc, buf0)

        @pl.when(r + 2 < BPW)
        def _():
            pltpu.async_copy(table_hbm.at[idx_v.at[r + 2]], buf0, sem0)

        pltpu.make_async_copy(table_hbm.at[idx_v.at[1]], buf1, sem1).wait()
        _accum(acc, buf1)

        @pl.when(r + 3 < BPW)
        def _():
            pltpu.async_copy(table_hbm.at[idx_v.at[r + 3]], buf1, sem1)

        return 0

    lax.fori_loop(0, BPW // 2, step, 0)

    pltpu.sync_copy(acc, shared.at[sidx], add=True)
    plsc.subcore_barrier()

    @pl.when(sid == 0)
    def _():
        pltpu.sync_copy(shared.at[pl.ds(0, HL)], buf0)
        scale = jnp.full((L,), 1.0 / B, jnp.float32)
        for l in range(HL):
            for c in range(NCH):
                sl = pl.ds(c * L, L)
                buf0[l, sl] = buf0[l, sl] * scale
        pltpu.sync_copy(buf0, out_hbm.at[cid])


@jax.jit
def kernel(x, table):
    mesh = plsc.VectorSubcoreMesh(core_axis_name="c", subcore_axis_name="s")
    partials = pl.kernel(
        _sc_body,
        out_type=jax.ShapeDtypeStruct((NC, HL, D), jnp.float32),
        mesh=mesh,
        compiler_params=pltpu.CompilerParams(use_tc_tiling_on_sc=False),
        scratch_types=[
            pltpu.VMEM((BPW, HL), jnp.int32),
            pltpu.VMEM((HL, D), jnp.float32),
            pltpu.VMEM((HL, D), jnp.float32),
            pltpu.VMEM((64, D), jnp.float32),
            pltpu.VMEM((64,), jnp.int32),
            pltpu.SemaphoreType.DMA,
            pltpu.SemaphoreType.DMA,
            pltpu.VMEM_SHARED((HL + 1, D), jnp.float32),
        ],
    )(x, table)
    return partials[0] + partials[1]


# speedup vs baseline: 2.9787x; 1.2769x over previous
import jax
import jax.numpy as jnp
from jax import lax
from jax.experimental import pallas as pl
from jax.experimental.pallas import tpu as pltpu
from jax.experimental.pallas import tpu_sc as plsc

NC = 2
NS = 16
NW = NC * NS
L = 16

B = 16384
HL = 50
D = 32
BPW = B // NW
NCH = D // L
LAG = 8


def _sc_body(x_hbm, table_hbm, out_hbm, idx_v, accg, acc, sidx, sem, shared):
    cid = lax.axis_index("c")
    sid = lax.axis_index("s")
    wid = sid * NC + cid
    base = wid * BPW

    pltpu.sync_copy(x_hbm.at[pl.ds(base, BPW)], idx_v)

    zero = jnp.zeros((L,), jnp.float32)
    for l in range(HL):
        for c in range(NCH):
            accg[l, pl.ds(c * L, L)] = zero
    for l in range(64):
        for c in range(NCH):
            acc[l, pl.ds(c * L, L)] = zero

    iota = lax.iota(jnp.int32, L)
    for c in range(4):
        v = iota + (c * L)
        sidx[pl.ds(c * L, L)] = jnp.where(v < HL, v, 0)

    @pl.when(sid == 0)
    def _():
        pltpu.sync_copy(acc.at[pl.ds(0, HL + 1)], shared)

    plsc.subcore_barrier()

    def step(i, _):
        pltpu.async_copy(table_hbm.at[idx_v.at[i]], accg, sem, add=True)

        @pl.when(i >= LAG)
        def _():
            pltpu.make_async_copy(table_hbm.at[idx_v.at[0]], accg, sem).wait()

        return 0

    lax.fori_loop(0, BPW, step, 0)
    for _ in range(LAG):
        pltpu.make_async_copy(table_hbm.at[idx_v.at[0]], accg, sem).wait()

    for l in range(HL):
        for c in range(NCH):
            sl = pl.ds(c * L, L)
            acc[l, sl] = accg[l, sl]

    pltpu.sync_copy(acc, shared.at[sidx], add=True)
    plsc.subcore_barrier()

    @pl.when(sid == 0)
    def _():
        pltpu.sync_copy(shared.at[pl.ds(0, HL)], accg)
        scale = jnp.full((L,), 1.0 / B, jnp.float32)
        for l in range(HL):
            for c in range(NCH):
                sl = pl.ds(c * L, L)
                accg[l, sl] = accg[l, sl] * scale
        pltpu.sync_copy(accg, out_hbm.at[cid])


@jax.jit
def kernel(x, table):
    mesh = plsc.VectorSubcoreMesh(core_axis_name="c", subcore_axis_name="s")
    partials = pl.kernel(
        _sc_body,
        out_type=jax.ShapeDtypeStruct((NC, HL, D), jnp.float32),
        mesh=mesh,
        compiler_params=pltpu.CompilerParams(use_tc_tiling_on_sc=False),
        scratch_types=[
            pltpu.VMEM((BPW, HL), jnp.int32),
            pltpu.VMEM((HL, D), jnp.float32),
            pltpu.VMEM((64, D), jnp.float32),
            pltpu.VMEM((64,), jnp.int32),
            pltpu.SemaphoreType.DMA,
            pltpu.VMEM_SHARED((HL + 1, D), jnp.float32),
        ],
    )(x, table)
    return partials[0] + partials[1]
